# baseline (speedup 1.0000x reference)
import functools

import jax
import jax.numpy as jnp
from jax import lax
from jax.experimental import pallas as pl
from jax.experimental.pallas import tpu as pltpu
from jax.experimental.pallas import tpu_sc as plsc

N = 10000
E = 320000
D = 128

NC = 2
NS = 16
NW = NC * NS
E_PER_W = E // NW
C = 80
CHUNKS = E_PER_W // C
U = 8
RB = 4
GD = 3
OUTER = 15
DC = 80
DCHUNKS = E_PER_W // DC
DOUTER = (DCHUNKS - 1) // 4
SLAB = 640
SLAB_TAIL = N - 15 * SLAB

_mesh = plsc.VectorSubcoreMesh(core_axis_name="c", subcore_axis_name="s")


@functools.partial(
    pl.kernel,
    out_type=jax.ShapeDtypeStruct((NC, N), jnp.float32),
    mesh=_mesh,
    scratch_types=[
        [pltpu.VMEM((DC,), jnp.int32)] * 4,
        pltpu.VMEM((DC,), jnp.float32),
        pltpu.VMEM_SHARED((N,), jnp.float32),
        [pltpu.SemaphoreType.DMA] * 4,
        [pltpu.SemaphoreType.DMA] * 2,
    ],
)
def _deg_kernel(dst_hbm, zeros_hbm, out_hbm, idx_v, ones_v, table,
                isem, ssem):
    cid = lax.axis_index("c")
    sid = lax.axis_index("s")
    wid = sid * NC + cid

    for i in range(DC // 16):
        ones_v[pl.ds(i * 16, 16)] = jnp.ones((16,), jnp.float32)

    @pl.when(sid == 0)
    def _():
        pltpu.sync_copy(zeros_hbm, table)

    plsc.subcore_barrier()
    base_w = wid * E_PER_W

    def fire_idx(jc, s):
        pltpu.async_copy(dst_hbm.at[pl.ds(base_w + jc * DC, DC)],
                         idx_v[s], isem[s])

    def wait_idx(jc, s):
        pltpu.make_async_copy(dst_hbm.at[pl.ds(base_w + jc * DC, DC)],
                              idx_v[s], isem[s]).wait()

    def fire_scatter(s, r):
        pltpu.async_copy(ones_v, table.at[idx_v[s]], ssem[r], add=True)

    def wait_scatter(s, r):
        pltpu.make_async_copy(ones_v, table.at[idx_v[s]], ssem[r]).wait()

    fire_idx(0, 0)
    fire_idx(1, 1)

    def body(j0, carry):
        for k in range(4):
            j = j0 * 4 + k
            if k >= 2:
                wait_scatter((k - 2) % 4, k % 2)
            else:
                @pl.when(j0 > 0)
                def _():
                    wait_scatter((k + 2) % 4, k % 2)
            if k == 3:
                @pl.when(j0 < DOUTER - 1)
                def _():
                    fire_idx(j + 2, (k + 2) % 4)
            else:
                fire_idx(j + 2, (k + 2) % 4)
            wait_idx(j, k)
            fire_scatter(k, k % 2)
        return carry

    lax.fori_loop(0, DOUTER, body, 0)
    wait_scatter(2, 0)
    wait_idx(DCHUNKS - 1, 0)
    fire_scatter(0, 0)
    wait_scatter(3, 1)
    wait_scatter(0, 0)
    plsc.subcore_barrier()

    @pl.when(sid == 0)
    def _():
        pltpu.sync_copy(table, out_hbm.at[cid])


@functools.partial(
    pl.kernel,
    out_type=jax.ShapeDtypeStruct((NC, N, D), jnp.float32),
    mesh=_mesh,
    scratch_types=[
        [pltpu.VMEM((C,), jnp.int32)] * U,
        [pltpu.VMEM((C,), jnp.int32)] * U,
        [pltpu.VMEM((C, D), jnp.float32)] * RB,
        pltpu.VMEM_SHARED((N, D), jnp.float32),
        [pltpu.SemaphoreType.DMA] * U,
        [pltpu.SemaphoreType.DMA] * RB,
        [pltpu.SemaphoreType.DMA] * RB,
    ],
)
def _agg_kernel(g_hbm, src_hbm, dst_hbm, zeros_hbm, out_hbm,
                src_v, dst_v, rows, acc, isem, gsem, ssem):
    cid = lax.axis_index("c")
    sid = lax.axis_index("s")
    wid = sid * NC + cid

    row0 = sid * SLAB

    def _init(nrows):
        @pl.when(cid == 0)
        def _():
            pltpu.sync_copy(g_hbm.at[pl.ds(row0, nrows)],
                            acc.at[pl.ds(row0, nrows)])

        @pl.when(cid == 1)
        def _():
            pltpu.sync_copy(zeros_hbm.at[pl.ds(row0, nrows)],
                            acc.at[pl.ds(row0, nrows)])

    @pl.when(sid < 15)
    def _():
        _init(SLAB)

    @pl.when(sid == 15)
    def _():
        _init(SLAB_TAIL)

    plsc.subcore_barrier()
    base_w = wid * E_PER_W

    def fire_idx(jc, s):
        base = base_w + jc * C
        pltpu.async_copy(src_hbm.at[pl.ds(base, C)], src_v[s], isem[s])
        pltpu.async_copy(dst_hbm.at[pl.ds(base, C)], dst_v[s], isem[s])

    def wait_idx(jc, s):
        base = base_w + jc * C
        pltpu.make_async_copy(src_hbm.at[pl.ds(base, C)], src_v[s],
                              isem[s]).wait()
        pltpu.make_async_copy(dst_hbm.at[pl.ds(base, C)], dst_v[s],
                              isem[s]).wait()

    def fire_gather(s, r):
        pltpu.async_copy(g_hbm.at[src_v[s]], rows[r], gsem[r])

    def wait_gather(s, r):
        pltpu.make_async_copy(g_hbm.at[src_v[s]], rows[r], gsem[r]).wait()

    def fire_scatter(s, r):
        pltpu.async_copy(rows[r], acc.at[dst_v[s]], ssem[r], add=True)

    def wait_scatter(s, r):
        pltpu.make_async_copy(rows[r], acc.at[dst_v[s]], ssem[r]).wait()

    for p in range(4):
        fire_idx(p, p)

    def body(j0, carry):
        for k in range(U):
            j = j0 * U + k
            if k >= 4:
                wait_scatter((k - 4) % U, k % RB)
            else:
                @pl.when(j0 > 0)
                def _():
                    wait_scatter((k + 4) % U, k % RB)
            fire_idx(j + 4, (k + 4) % U)
            wait_idx(j, k)
            fire_gather(k, k % RB)
            if k >= 3:
                wait_gather(k - 3, (k - 3) % RB)
                fire_scatter(k - 3, (k - 3) % RB)
            else:
                @pl.when(j0 > 0)
                def _():
                    wait_gather((k + 5) % U, (k + 1) % RB)
                    fire_scatter((k + 5) % U, (k + 1) % RB)
        return carry

    lax.fori_loop(0, OUTER, body, 0)
    wait_scatter(4, 0)
    fire_idx(CHUNKS - 1, 4)
    wait_idx(120, 0)
    fire_gather(0, 0)
    wait_gather(5, 1)
    fire_scatter(5, 1)
    wait_scatter(5, 1)
    wait_idx(121, 1)
    fire_gather(1, 1)
    wait_gather(6, 2)
    fire_scatter(6, 2)
    wait_scatter(6, 2)
    wait_idx(122, 2)
    fire_gather(2, 2)
    wait_gather(7, 3)
    fire_scatter(7, 3)
    wait_scatter(7, 3)
    wait_idx(123, 3)
    fire_gather(3, 3)
    wait_gather(0, 0)
    fire_scatter(0, 0)
    wait_scatter(0, 0)
    wait_idx(124, 4)
    fire_gather(4, 0)
    wait_gather(1, 1)
    fire_scatter(1, 1)
    wait_gather(2, 2)
    fire_scatter(2, 2)
    wait_gather(3, 3)
    fire_scatter(3, 3)
    wait_gather(4, 0)
    fire_scatter(4, 0)
    wait_scatter(1, 1)
    wait_scatter(2, 2)
    wait_scatter(3, 3)
    wait_scatter(4, 0)
    plsc.subcore_barrier()

    @pl.when(sid < 15)
    def _():
        pltpu.sync_copy(acc.at[pl.ds(row0, SLAB)],
                        out_hbm.at[cid, pl.ds(row0, SLAB)])

    @pl.when(sid == 15)
    def _():
        pltpu.sync_copy(acc.at[pl.ds(row0, SLAB_TAIL)],
                        out_hbm.at[cid, pl.ds(row0, SLAB_TAIL)])


_R = 2000
_GRID = N // _R


def _first_body(x_ref, w_ref, deg_ref, out_ref):
    d = deg_ref[0] + deg_ref[1] + 1.0
    dinv = lax.rsqrt(d)
    h = jnp.dot(x_ref[...], w_ref[...], preferred_element_type=jnp.float32)
    out_ref[...] = h * dinv


def _mid_body(p_ref, deg_ref, b_ref, w_ref, out_ref):
    d = deg_ref[0] + deg_ref[1] + 1.0
    dinv = lax.rsqrt(d)
    s = (p_ref[0] + p_ref[1]) * dinv + b_ref[...][None, :]
    y = jnp.maximum(s, 0.0)
    h = jnp.dot(y, w_ref[...], preferred_element_type=jnp.float32)
    out_ref[...] = h * dinv


def _last_body(p_ref, deg_ref, b_ref, w_ref, bp_ref, out_ref):
    d = deg_ref[0] + deg_ref[1] + 1.0
    dinv = lax.rsqrt(d)
    s = (p_ref[0] + p_ref[1]) * dinv + b_ref[...][None, :]
    y = jnp.maximum(s, 0.0)
    h = jnp.dot(y, w_ref[...], preferred_element_type=jnp.float32)
    out_ref[...] = h + bp_ref[...][None, :]


_row_spec = pl.BlockSpec((_R, D), lambda i: (i, 0))
_pair_spec = pl.BlockSpec((NC, _R, D), lambda i: (0, i, 0))
_deg_spec = pl.BlockSpec((NC, _R, 1), lambda i: (0, i, 0))
_w_spec = pl.BlockSpec((D, D), lambda i: (0, 0))
_b_spec = pl.BlockSpec((D,), lambda i: (0,))
_out_sds = jax.ShapeDtypeStruct((N, D), jnp.float32)

_first_tc = pl.pallas_call(
    _first_body, grid=(_GRID,),
    in_specs=[_row_spec, _w_spec, _deg_spec],
    out_specs=_row_spec, out_shape=_out_sds)

_mid_tc = pl.pallas_call(
    _mid_body, grid=(_GRID,),
    in_specs=[_pair_spec, _deg_spec, _b_spec, _w_spec],
    out_specs=_row_spec, out_shape=_out_sds)

_last_tc = pl.pallas_call(
    _last_body, grid=(_GRID,),
    in_specs=[_pair_spec, _deg_spec, _b_spec, _w_spec, _b_spec],
    out_specs=_row_spec, out_shape=_out_sds)


def kernel(x, edge_index, W0, b0, W1, b1, W2, b2, Wp, bp):
    src = edge_index[0].astype(jnp.int32)
    dst = edge_index[1].astype(jnp.int32)
    dst_flat = dst
    zeros_nd = jnp.zeros((N, D), jnp.float32)
    zeros_n = jnp.zeros((N,), jnp.float32)

    deg_p = _deg_kernel(dst_flat, zeros_n)
    deg = deg_p.reshape(NC, N, 1)

    g0 = _first_tc(x, W0, deg)
    p1 = _agg_kernel(g0, src, dst, zeros_nd)
    g1 = _mid_tc(p1, deg, b0, W1)
    p2 = _agg_kernel(g1, src, dst, zeros_nd)
    g2 = _mid_tc(p2, deg, b1, W2)
    p3 = _agg_kernel(g2, src, dst, zeros_nd)
    return _last_tc(p3, deg, b2, Wp, bp)


# speedup vs baseline: 1.0023x; 1.0023x over previous
import functools

import jax
import jax.numpy as jnp
from jax import lax
from jax.experimental import pallas as pl
from jax.experimental.pallas import tpu as pltpu
from jax.experimental.pallas import tpu_sc as plsc

N = 10000
E = 320000
D = 128

NC = 2
NS = 16
NW = NC * NS
E_PER_W = E // NW
C = 80
CHUNKS = E_PER_W // C
U = 8
RB = 4
OUTER = 15
DC = 80
DCHUNKS = E_PER_W // DC
DOUTER = (DCHUNKS - 1) // 4
SLAB = 640
SLAB_TAIL = N - 15 * SLAB

_mesh = plsc.VectorSubcoreMesh(core_axis_name="c", subcore_axis_name="s")


@functools.partial(
    pl.kernel,
    out_type=jax.ShapeDtypeStruct((NC, N), jnp.float32),
    mesh=_mesh,
    scratch_types=[
        [pltpu.VMEM((DC,), jnp.int32)] * 4,
        pltpu.VMEM((DC,), jnp.float32),
        pltpu.VMEM_SHARED((N,), jnp.float32),
        [pltpu.SemaphoreType.DMA] * 4,
        [pltpu.SemaphoreType.DMA] * 2,
    ],
)
def _deg_kernel(dst_hbm, zeros_hbm, out_hbm, idx_v, ones_v, table,
                isem, ssem):
    cid = lax.axis_index("c")
    sid = lax.axis_index("s")
    wid = sid * NC + cid

    for i in range(DC // 16):
        ones_v[pl.ds(i * 16, 16)] = jnp.ones((16,), jnp.float32)

    @pl.when(sid == 0)
    def _():
        pltpu.sync_copy(zeros_hbm, table)

    plsc.subcore_barrier()
    base_w = wid * E_PER_W

    def fire_idx(jc, s):
        pltpu.async_copy(dst_hbm.at[pl.ds(base_w + jc * DC, DC)],
                         idx_v[s], isem[s])

    def wait_idx(jc, s):
        pltpu.make_async_copy(dst_hbm.at[pl.ds(base_w + jc * DC, DC)],
                              idx_v[s], isem[s]).wait()

    def fire_scatter(s, r):
        pltpu.async_copy(ones_v, table.at[idx_v[s]], ssem[r], add=True)

    def wait_scatter(s, r):
        pltpu.make_async_copy(ones_v, table.at[idx_v[s]], ssem[r]).wait()

    fire_idx(0, 0)
    fire_idx(1, 1)

    def body(j0, carry):
        for k in range(4):
            j = j0 * 4 + k
            if k >= 2:
                wait_scatter((k - 2) % 4, k % 2)
            else:
                @pl.when(j0 > 0)
                def _():
                    wait_scatter((k + 2) % 4, k % 2)
            if k == 3:
                @pl.when(j0 < DOUTER - 1)
                def _():
                    fire_idx(j + 2, (k + 2) % 4)
            else:
                fire_idx(j + 2, (k + 2) % 4)
            wait_idx(j, k)
            fire_scatter(k, k % 2)
        return carry

    lax.fori_loop(0, DOUTER, body, 0)
    wait_scatter(2, 0)
    wait_idx(DCHUNKS - 1, 0)
    fire_scatter(0, 0)
    wait_scatter(3, 1)
    wait_scatter(0, 0)
    plsc.subcore_barrier()

    @pl.when(sid == 0)
    def _():
        pltpu.sync_copy(table, out_hbm.at[cid])


@functools.partial(
    pl.kernel,
    out_type=jax.ShapeDtypeStruct((NC, N, D), jnp.float32),
    mesh=_mesh,
    scratch_types=[
        [pltpu.VMEM((C,), jnp.int32)] * U,
        [pltpu.VMEM((C,), jnp.int32)] * U,
        [pltpu.VMEM((C, D), jnp.float32)] * RB,
        pltpu.VMEM_SHARED((N, D), jnp.float32),
        [pltpu.SemaphoreType.DMA] * U,
        [pltpu.SemaphoreType.DMA] * RB,
        [pltpu.SemaphoreType.DMA] * RB,
    ],
)
def _agg_kernel(g_hbm, src_hbm, dst_hbm, zeros_hbm, out_hbm,
                src_v, dst_v, rows, acc, isem, gsem, ssem):
    cid = lax.axis_index("c")
    sid = lax.axis_index("s")
    wid = sid * NC + cid

    row0 = sid * SLAB

    def _init(nrows):
        @pl.when(cid == 0)
        def _():
            pltpu.sync_copy(g_hbm.at[pl.ds(row0, nrows)],
                            acc.at[pl.ds(row0, nrows)])

        @pl.when(cid == 1)
        def _():
            pltpu.sync_copy(zeros_hbm.at[pl.ds(row0, nrows)],
                            acc.at[pl.ds(row0, nrows)])

    @pl.when(sid < 15)
    def _():
        _init(SLAB)

    @pl.when(sid == 15)
    def _():
        _init(SLAB_TAIL)

    plsc.subcore_barrier()
    base_w = wid * E_PER_W

    def fire_idx(jc, s):
        base = base_w + jc * C
        pltpu.async_copy(src_hbm.at[pl.ds(base, C)], src_v[s], isem[s])
        pltpu.async_copy(dst_hbm.at[pl.ds(base, C)], dst_v[s], isem[s])

    def wait_idx(jc, s):
        base = base_w + jc * C
        pltpu.make_async_copy(src_hbm.at[pl.ds(base, C)], src_v[s],
                              isem[s]).wait()
        pltpu.make_async_copy(dst_hbm.at[pl.ds(base, C)], dst_v[s],
                              isem[s]).wait()

    def fire_gather(s, r):
        pltpu.async_copy(g_hbm.at[src_v[s]], rows[r], gsem[r])

    def wait_gather(s, r):
        pltpu.make_async_copy(g_hbm.at[src_v[s]], rows[r], gsem[r]).wait()

    def fire_scatter(s, r):
        pltpu.async_copy(rows[r], acc.at[dst_v[s]], ssem[r], add=True)

    def wait_scatter(s, r):
        pltpu.make_async_copy(rows[r], acc.at[dst_v[s]], ssem[r]).wait()

    for p in range(4):
        fire_idx(p, p)

    def body(j0, carry):
        for k in range(U):
            j = j0 * U + k
            if k >= 4:
                wait_scatter((k - 4) % U, k % RB)
            else:
                @pl.when(j0 > 0)
                def _():
                    wait_scatter((k + 4) % U, k % RB)
            fire_idx(j + 4, (k + 4) % U)
            wait_idx(j, k)
            fire_gather(k, k % RB)
            if k >= 3:
                wait_gather(k - 3, (k - 3) % RB)
                fire_scatter(k - 3, (k - 3) % RB)
            else:
                @pl.when(j0 > 0)
                def _():
                    wait_gather((k + 5) % U, (k + 1) % RB)
                    fire_scatter((k + 5) % U, (k + 1) % RB)
        return carry

    lax.fori_loop(0, OUTER, body, 0)
    wait_scatter(4, 0)
    fire_idx(CHUNKS - 1, 4)
    wait_idx(120, 0)
    fire_gather(0, 0)
    wait_gather(5, 1)
    fire_scatter(5, 1)
    wait_scatter(5, 1)
    wait_idx(121, 1)
    fire_gather(1, 1)
    wait_gather(6, 2)
    fire_scatter(6, 2)
    wait_scatter(6, 2)
    wait_idx(122, 2)
    fire_gather(2, 2)
    wait_gather(7, 3)
    fire_scatter(7, 3)
    wait_scatter(7, 3)
    wait_idx(123, 3)
    fire_gather(3, 3)
    wait_gather(0, 0)
    fire_scatter(0, 0)
    wait_scatter(0, 0)
    wait_idx(124, 4)
    fire_gather(4, 0)
    wait_gather(1, 1)
    fire_scatter(1, 1)
    wait_gather(2, 2)
    fire_scatter(2, 2)
    wait_gather(3, 3)
    fire_scatter(3, 3)
    wait_gather(4, 0)
    fire_scatter(4, 0)
    wait_scatter(1, 1)
    wait_scatter(2, 2)
    wait_scatter(3, 3)
    wait_scatter(4, 0)
    plsc.subcore_barrier()

    @pl.when(sid < 15)
    def _():
        pltpu.sync_copy(acc.at[pl.ds(row0, SLAB)],
                        out_hbm.at[cid, pl.ds(row0, SLAB)])

    @pl.when(sid == 15)
    def _():
        pltpu.sync_copy(acc.at[pl.ds(row0, SLAB_TAIL)],
                        out_hbm.at[cid, pl.ds(row0, SLAB_TAIL)])


_R = 2000
_GRID = N // _R


def _first_body(x_ref, w_ref, deg_ref, out_ref):
    d = deg_ref[0] + deg_ref[1] + 1.0
    dinv = lax.rsqrt(d)
    h = jnp.dot(x_ref[...], w_ref[...], preferred_element_type=jnp.float32)
    out_ref[...] = h * dinv


def _mid_body(p_ref, deg_ref, b_ref, w_ref, out_ref):
    d = deg_ref[0] + deg_ref[1] + 1.0
    dinv = lax.rsqrt(d)
    s = (p_ref[0] + p_ref[1]) * dinv + b_ref[...][None, :]
    y = jnp.maximum(s, 0.0)
    h = jnp.dot(y, w_ref[...], preferred_element_type=jnp.float32)
    out_ref[...] = h * dinv


def _last_body(p_ref, deg_ref, b_ref, w_ref, bp_ref, out_ref):
    d = deg_ref[0] + deg_ref[1] + 1.0
    dinv = lax.rsqrt(d)
    s = (p_ref[0] + p_ref[1]) * dinv + b_ref[...][None, :]
    y = jnp.maximum(s, 0.0)
    h = jnp.dot(y, w_ref[...], preferred_element_type=jnp.float32)
    out_ref[...] = h + bp_ref[...][None, :]


_row_spec = pl.BlockSpec((_R, D), lambda i: (i, 0))
_pair_spec = pl.BlockSpec((NC, _R, D), lambda i: (0, i, 0))
_deg_spec = pl.BlockSpec((NC, _R, 1), lambda i: (0, i, 0))
_w_spec = pl.BlockSpec((D, D), lambda i: (0, 0))
_b_spec = pl.BlockSpec((D,), lambda i: (0,))
_out_sds = jax.ShapeDtypeStruct((N, D), jnp.float32)

_first_tc = pl.pallas_call(
    _first_body, grid=(_GRID,),
    in_specs=[_row_spec, _w_spec, _deg_spec],
    out_specs=_row_spec, out_shape=_out_sds)

_mid_tc = pl.pallas_call(
    _mid_body, grid=(_GRID,),
    in_specs=[_pair_spec, _deg_spec, _b_spec, _w_spec],
    out_specs=_row_spec, out_shape=_out_sds)

_last_tc = pl.pallas_call(
    _last_body, grid=(_GRID,),
    in_specs=[_pair_spec, _deg_spec, _b_spec, _w_spec, _b_spec],
    out_specs=_row_spec, out_shape=_out_sds)


def kernel(x, edge_index, W0, b0, W1, b1, W2, b2, Wp, bp):
    src = edge_index[0].astype(jnp.int32)
    dst = edge_index[1].astype(jnp.int32)
    zeros_nd = jnp.zeros((N, D), jnp.float32)
    zeros_n = jnp.zeros((N,), jnp.float32)

    deg_p = _deg_kernel(dst, zeros_n)
    deg = deg_p.reshape(NC, N, 1)

    g0 = _first_tc(x, W0, deg)
    p1 = _agg_kernel(g0, src, dst, zeros_nd)
    g1 = _mid_tc(p1, deg, b0, W1)
    p2 = _agg_kernel(g1, src, dst, zeros_nd)
    g2 = _mid_tc(p2, deg, b1, W2)
    p3 = _agg_kernel(g2, src, dst, zeros_nd)
    return _last_tc(p3, deg, b2, Wp, bp)
